# baseline (speedup 1.0000x reference)
import functools

import jax
import jax.numpy as jnp
from jax import lax
from jax.experimental import pallas as pl
from jax.experimental.pallas import tpu as pltpu
from jax.experimental.pallas import tpu_sc as plsc

_NC = 2
_NS = 16
_NW = _NC * _NS
_L = 16
_BB = 128


def _embed_body(seq, embed, idx_hbm, table_hbm, pos_hbm, out_hbm,
                idx_v, pos_v, rows_v, obuf_v, gs0, gs1, os0, os1):
    gsem = (gs0, gs1)
    osem = (os0, os1)
    nvec = embed // _L
    wid = lax.axis_index("s") * _NC + lax.axis_index("c")

    pltpu.sync_copy(idx_hbm.at[:, pl.ds(wid * _BB, _BB)], idx_v)
    pltpu.sync_copy(pos_hbm.at[pl.ds(0, seq)], pos_v)

    iota = lax.iota(jnp.int32, _L)
    sc_idx = []
    for c in range(nvec):
        e = iota + _L * c
        sc_idx.append((e // 8, e % 8))

    def fire(s, b):
        def vec_body(v, carry):
            vec = idx_v[s, pl.ds(v * _L, _L)]
            for lane in range(_L):
                tok = vec[lane]
                pltpu.async_copy(
                    table_hbm.at[pl.ds(tok, 1)],
                    rows_v.at[b, pl.ds(v * _L + lane, 1)],
                    gsem[b],
                )
            return carry
        lax.fori_loop(0, _BB // _L, vec_body, 0)

    def process(s, b, wait_out):
        pltpu.make_async_copy(
            table_hbm.at[pl.ds(0, _BB)], rows_v.at[b], gsem[b]).wait()
        if wait_out:
            pltpu.make_async_copy(
                obuf_v.at[b], out_hbm.at[0, :, pl.ds(0, 8)], osem[b]).wait()
        p = [pos_v[s, pl.ds(_L * c, _L)] for c in range(nvec)]

        def tok_body(t, carry):
            tvec = lax.broadcast(t, (_L,))
            for c in range(nvec):
                g = rows_v[b, t, pl.ds(_L * c, _L)]
                plsc.store_scatter(
                    obuf_v.at[b], [sc_idx[c][0], sc_idx[c][1], tvec], g + p[c])
            return carry

        lax.fori_loop(0, _BB, tok_body, 0, unroll=2)
        pltpu.async_copy(
            obuf_v.at[b], out_hbm.at[s, :, pl.ds(wid * 8, 8)], osem[b])

    fire(0, 0)
    fire(1, 1)
    process(0, 0, False)
    fire(2, 0)
    process(1, 1, False)
    fire(3, 1)

    def loop_body(j, carry):
        for b in range(2):
            k = 2 * j + 2 + b
            process(k, b, True)
            fire(k + 2, b)
        return carry

    lax.fori_loop(0, (seq - 4) // 2, loop_body, 0)

    process(seq - 2, 0, True)
    process(seq - 1, 1, True)
    pltpu.make_async_copy(
        obuf_v.at[0], out_hbm.at[0, :, pl.ds(0, 8)], os0).wait()
    pltpu.make_async_copy(
        obuf_v.at[1], out_hbm.at[0, :, pl.ds(0, 8)], os1).wait()


def kernel(token_ids, text_table, pos_table):
    batch, seq = token_ids.shape
    vocab, embed = text_table.shape
    tok_t = token_ids.T.astype(jnp.int32)

    mesh = plsc.VectorSubcoreMesh(core_axis_name="c", subcore_axis_name="s")
    body = functools.partial(_embed_body, seq, embed)
    out4 = pl.kernel(
        body,
        out_type=jax.ShapeDtypeStruct(
            (seq, embed // 8, (batch // _BB) * 8, _BB), jnp.float32),
        mesh=mesh,
        scratch_types=[
            pltpu.VMEM((seq, _BB), jnp.int32),
            pltpu.VMEM((seq, embed), jnp.float32),
            pltpu.VMEM((2, _BB, embed), jnp.float32),
            pltpu.VMEM((2, embed // 8, 8, _BB), jnp.float32),
            pltpu.SemaphoreType.DMA,
            pltpu.SemaphoreType.DMA,
            pltpu.SemaphoreType.DMA,
            pltpu.SemaphoreType.DMA,
        ],
        compiler_params=pltpu.CompilerParams(
            use_tc_tiling_on_sc=True, needs_layout_passes=False),
        name="sc_embed_lookup",
    )(tok_t, text_table, pos_table)
    out = out4.reshape(seq, embed // 8, batch // _BB, 8, _BB)
    return out.transpose(2, 4, 0, 1, 3).reshape(batch, seq, embed)
